# Initial kernel scaffold; baseline (speedup 1.0000x reference)
import jax
import jax.numpy as jnp
from jax.experimental import pallas as pl


def kernel(x, edge_index, W1, b1, W2, b2):
    raise NotImplementedError("write your pallas kernel here")


# speedup vs baseline: 3.8037x; 3.8037x over previous
import jax
import jax.numpy as jnp
from jax import lax
from jax.experimental import pallas as pl
from jax.experimental.pallas import tpu as pltpu
from jax.experimental.pallas import tpu_sc as plsc

N = 10000
E = 320000
D_IN = 128
D_HID = 128
D_OUT = 64

NROW = 10240
NC, NS = 2, 16
NW = NC * NS
CH = 128
NCHUNK = 80
EPW = CH * NCHUNK
EPAD = EPW * NW
HR = 256
HDST = 128
DEG_RPT = HR // NS


def _deg_body(src_hbm, dst_hbm, out_hbm, sidx, didx, hist, ria, rib, zb, acc):
    cid = lax.axis_index("c")
    sid = lax.axis_index("s")
    wid = sid * NC + cid
    zeros16 = jnp.zeros((16,), jnp.float32)
    ones16 = jnp.ones((16,), jnp.float32)
    iota16 = lax.iota(jnp.int32, 16)

    def zh(r, _):
        for c in range(8):
            hist[r, pl.ds(c * 16, 16)] = zeros16
        return 0

    lax.fori_loop(0, HR, zh, 0)

    for i in range(5):
        ria[pl.ds(i * 16, 16)] = iota16 + i * 16
        rib[pl.ds(i * 16, 16)] = iota16 + (HDST + i * 16)

    for r in range(DEG_RPT):
        for c in range(8):
            zb[r, pl.ds(c * 16, 16)] = zeros16
    pltpu.sync_copy(zb, acc.at[pl.ds(sid * DEG_RPT, DEG_RPT)])

    pltpu.sync_copy(src_hbm.at[pl.ds(wid * EPW, EPW)], sidx)
    pltpu.sync_copy(dst_hbm.at[pl.ds(wid * EPW, EPW)], didx)

    def step(i, _):
        s16 = sidx[pl.ds(i * 16, 16)]
        plsc.addupdate_scatter(
            hist,
            [lax.shift_right_logical(s16, 7), lax.bitwise_and(s16, 127)],
            ones16,
        )
        d16 = didx[pl.ds(i * 16, 16)]
        plsc.addupdate_scatter(
            hist,
            [lax.shift_right_logical(d16, 7) + HDST, lax.bitwise_and(d16, 127)],
            ones16,
        )
        return 0

    lax.fori_loop(0, EPW // 16, step, 0)

    plsc.subcore_barrier()
    pltpu.sync_copy(hist.at[pl.ds(0, 80)], acc.at[ria], add=True)
    pltpu.sync_copy(hist.at[pl.ds(HDST, 80)], acc.at[rib], add=True)
    plsc.subcore_barrier()

    sl = pl.ds(sid * DEG_RPT, DEG_RPT)
    pltpu.sync_copy(acc.at[sl], zb)
    pltpu.sync_copy(zb, out_hbm.at[cid, sl])


def _sc_deg(srcp, dstp):
    mesh = plsc.VectorSubcoreMesh(core_axis_name="c", subcore_axis_name="s")
    f = pl.kernel(
        _deg_body,
        out_type=jax.ShapeDtypeStruct((NC, HR, 128), jnp.float32),
        mesh=mesh,
        compiler_params=pltpu.CompilerParams(needs_layout_passes=False),
        scratch_types=[
            pltpu.VMEM((EPW,), jnp.int32),
            pltpu.VMEM((EPW,), jnp.int32),
            pltpu.VMEM((HR, 128), jnp.float32),
            pltpu.VMEM((80,), jnp.int32),
            pltpu.VMEM((80,), jnp.int32),
            pltpu.VMEM((DEG_RPT, 128), jnp.float32),
            pltpu.VMEM_SHARED((HR, 128), jnp.float32),
        ],
    )
    return f(srcp, dstp)


def _make_agg_body(dim):
    rpt = NROW // NS

    def body(tab_hbm, src_hbm, dst_hbm, out_hbm, sidx, didx, rows, acc, sem):
        cid = lax.axis_index("c")
        sid = lax.axis_index("s")
        wid = sid * NC + cid
        zeros16 = jnp.zeros((16,), jnp.float32)

        def zr(r, _):
            for c in range(dim // 16):
                rows[r, pl.ds(c * 16, 16)] = zeros16
            return 0

        lax.fori_loop(0, CH, zr, 0)
        for k in range(rpt // CH):
            pltpu.sync_copy(rows, acc.at[pl.ds(sid * rpt + k * CH, CH)])
        plsc.subcore_barrier()

        base = wid * EPW

        def step(c, _):
            off = base + c * CH
            pltpu.sync_copy(src_hbm.at[pl.ds(off, CH)], sidx)
            pltpu.sync_copy(dst_hbm.at[pl.ds(off, CH)], didx)
            pltpu.async_copy(tab_hbm.at[sidx], rows, sem).wait()
            pltpu.sync_copy(rows, acc.at[didx], add=True)
            return 0

        lax.fori_loop(0, NCHUNK, step, 0)

        plsc.subcore_barrier()
        for k in range(rpt // CH):
            sl = pl.ds(sid * rpt + k * CH, CH)
            pltpu.sync_copy(acc.at[sl], rows)
            pltpu.sync_copy(rows, out_hbm.at[cid, sl])

    return body


def _sc_agg(tab, srcp, dstp, dim):
    mesh = plsc.VectorSubcoreMesh(core_axis_name="c", subcore_axis_name="s")
    f = pl.kernel(
        _make_agg_body(dim),
        out_type=jax.ShapeDtypeStruct((NC, NROW, dim), jnp.float32),
        mesh=mesh,
        compiler_params=pltpu.CompilerParams(
            needs_layout_passes=False, use_tc_tiling_on_sc=False
        ),
        scratch_types=[
            pltpu.VMEM((CH,), jnp.int32),
            pltpu.VMEM((CH,), jnp.int32),
            pltpu.VMEM((CH, dim), jnp.float32),
            pltpu.VMEM_SHARED((NROW, dim), jnp.float32),
            pltpu.SemaphoreType.DMA,
        ],
    )
    return f(tab, srcp, dstp)


def _tc_prep(x_pad, dps, dpd):
    def body(x_ref, dps_ref, dpd_ref, t1_ref, ns_ref, nd_ref):
        ds_ = dps_ref[0] + dps_ref[1]
        dd = dpd_ref[0] + dpd_ref[1]
        ns = jnp.where(ds_ > 0, lax.rsqrt(ds_), 0.0)
        nd = jnp.where(dd > 0, lax.rsqrt(dd), 0.0)
        ns_ref[...] = ns
        nd_ref[...] = nd
        t1_ref[...] = x_ref[...] * ns

    return pl.pallas_call(
        body,
        out_shape=[
            jax.ShapeDtypeStruct((NROW, D_IN), jnp.float32),
            jax.ShapeDtypeStruct((NROW, 1), jnp.float32),
            jax.ShapeDtypeStruct((NROW, 1), jnp.float32),
        ],
    )(x_pad, dps, dpd)


def _tc_mid(p1, ns_col, nd_col, W1, b1r, W2):
    def body(p_ref, ns_ref, nd_ref, w1_ref, b1_ref, w2_ref, t2_ref):
        agg = p_ref[0] + p_ref[1]
        aggn = agg * nd_ref[...]
        h = jnp.dot(aggn, w1_ref[...], preferred_element_type=jnp.float32)
        h = jnp.maximum(h + b1_ref[...], 0.0)
        t2_ref[...] = jnp.dot(
            h * ns_ref[...], w2_ref[...], preferred_element_type=jnp.float32
        )

    return pl.pallas_call(
        body,
        out_shape=jax.ShapeDtypeStruct((NROW, D_OUT), jnp.float32),
    )(p1, ns_col, nd_col, W1, b1r, W2)


def _tc_final(p2, nd_col, b2r):
    def body(p_ref, nd_ref, b2_ref, o_ref):
        o_ref[...] = (p_ref[0, :N] + p_ref[1, :N]) * nd_ref[:N] + b2_ref[...]

    return pl.pallas_call(
        body,
        out_shape=jax.ShapeDtypeStruct((N, D_OUT), jnp.float32),
    )(p2, nd_col, b2r)


def kernel(x, edge_index, W1, b1, W2, b2):
    src = edge_index[0]
    dst = edge_index[1]
    padi = jnp.full((EPAD - E,), N, jnp.int32)
    srcp = jnp.concatenate([src, padi])
    dstp = jnp.concatenate([dst, padi])
    x_pad = jnp.pad(x, ((0, NROW - N), (0, 0)))

    degp = _sc_deg(srcp, dstp)
    dps = degp[:, :80, :].reshape(NC, NROW, 1)
    dpd = degp[:, HDST:HDST + 80, :].reshape(NC, NROW, 1)
    t1, ns_col, nd_col = _tc_prep(x_pad, dps, dpd)

    p1 = _sc_agg(t1, srcp, dstp, D_HID)
    t2 = _tc_mid(p1, ns_col, nd_col, W1, b1.reshape(1, D_HID), W2)

    p2 = _sc_agg(t2, srcp, dstp, D_OUT)
    return _tc_final(p2, nd_col, b2.reshape(1, D_OUT))


# speedup vs baseline: 5.0944x; 1.3393x over previous
import jax
import jax.numpy as jnp
from jax import lax
from jax.experimental import pallas as pl
from jax.experimental.pallas import tpu as pltpu
from jax.experimental.pallas import tpu_sc as plsc

N = 10000
E = 320000
D_IN = 128
D_HID = 128
D_OUT = 64

NROW = 10240
NC, NS = 2, 16
NW = NC * NS
CH = 128
NCHUNK = 80
EPW = CH * NCHUNK
EPAD = EPW * NW
HR = 256
HDST = 128
DEG_RPT = HR // NS


def _deg_body(src_hbm, dst_hbm, out_hbm, sidx, didx, hist, ria, rib, zb, acc):
    cid = lax.axis_index("c")
    sid = lax.axis_index("s")
    wid = sid * NC + cid
    zeros16 = jnp.zeros((16,), jnp.float32)
    ones16 = jnp.ones((16,), jnp.float32)
    iota16 = lax.iota(jnp.int32, 16)

    def zh(r, _):
        for c in range(8):
            hist[r, pl.ds(c * 16, 16)] = zeros16
        return 0

    lax.fori_loop(0, HR, zh, 0)

    for i in range(5):
        ria[pl.ds(i * 16, 16)] = iota16 + i * 16
        rib[pl.ds(i * 16, 16)] = iota16 + (HDST + i * 16)

    for r in range(DEG_RPT):
        for c in range(8):
            zb[r, pl.ds(c * 16, 16)] = zeros16
    pltpu.sync_copy(zb, acc.at[pl.ds(sid * DEG_RPT, DEG_RPT)])

    pltpu.sync_copy(src_hbm.at[pl.ds(wid * EPW, EPW)], sidx)
    pltpu.sync_copy(dst_hbm.at[pl.ds(wid * EPW, EPW)], didx)

    def step(i, _):
        s16 = sidx[pl.ds(i * 16, 16)]
        plsc.addupdate_scatter(
            hist,
            [lax.shift_right_logical(s16, 7), lax.bitwise_and(s16, 127)],
            ones16,
        )
        d16 = didx[pl.ds(i * 16, 16)]
        plsc.addupdate_scatter(
            hist,
            [lax.shift_right_logical(d16, 7) + HDST, lax.bitwise_and(d16, 127)],
            ones16,
        )
        return 0

    lax.fori_loop(0, EPW // 16, step, 0)

    plsc.subcore_barrier()
    pltpu.sync_copy(hist.at[pl.ds(0, 80)], acc.at[ria], add=True)
    pltpu.sync_copy(hist.at[pl.ds(HDST, 80)], acc.at[rib], add=True)
    plsc.subcore_barrier()

    sl = pl.ds(sid * DEG_RPT, DEG_RPT)
    pltpu.sync_copy(acc.at[sl], zb)
    pltpu.sync_copy(zb, out_hbm.at[cid, sl])


def _sc_deg(srcp, dstp):
    mesh = plsc.VectorSubcoreMesh(core_axis_name="c", subcore_axis_name="s")
    f = pl.kernel(
        _deg_body,
        out_type=jax.ShapeDtypeStruct((NC, HR, 128), jnp.float32),
        mesh=mesh,
        compiler_params=pltpu.CompilerParams(needs_layout_passes=False),
        scratch_types=[
            pltpu.VMEM((EPW,), jnp.int32),
            pltpu.VMEM((EPW,), jnp.int32),
            pltpu.VMEM((HR, 128), jnp.float32),
            pltpu.VMEM((80,), jnp.int32),
            pltpu.VMEM((80,), jnp.int32),
            pltpu.VMEM((DEG_RPT, 128), jnp.float32),
            pltpu.VMEM_SHARED((HR, 128), jnp.float32),
        ],
    )
    return f(srcp, dstp)


IXR = 8


def _make_agg_body(dim):
    rpt = NROW // NS

    def body(tab_hbm, src_hbm, dst2d_hbm, out_hbm, sidxb, didx, rows, acc,
             sem_i, sem_g, sem_s):
        cid = lax.axis_index("c")
        sid = lax.axis_index("s")
        wid = sid * NC + cid
        zeros16 = jnp.zeros((16,), jnp.float32)
        base = wid * EPW

        def stage_idx(c, j):
            pltpu.async_copy(src_hbm.at[pl.ds(base + c * CH, CH)], sidxb[j],
                             sem_i[j])

        def wait_idx(j):
            pltpu.make_async_copy(src_hbm.at[pl.ds(base, CH)], sidxb[j],
                                  sem_i[j]).wait()

        def start_gather(c, j, b):
            pltpu.async_copy(tab_hbm.at[sidxb[j]], rows[b], sem_g[b])

        def wait_gather(b):
            pltpu.make_async_copy(tab_hbm.at[sidxb[0]], rows[b],
                                  sem_g[b]).wait()

        def start_scatter(c, b):
            pltpu.async_copy(rows[b], acc.at[didx.at[c]], sem_s[b], add=True)

        def wait_scatter(b):
            pltpu.make_async_copy(rows[b], acc.at[didx.at[0]], sem_s[b]).wait()

        pltpu.sync_copy(dst2d_hbm.at[pl.ds(wid * NCHUNK, NCHUNK)], didx)
        for c in range(3):
            stage_idx(c, c)
        wait_idx(0)
        start_gather(0, 0, 0)

        def zrow(r, _):
            for c in range(dim // 16):
                rows[1][r, pl.ds(c * 16, 16)] = zeros16
            return 0

        lax.fori_loop(0, CH, zrow, 0)
        for k in range(rpt // CH):
            pltpu.sync_copy(rows[1], acc.at[pl.ds(sid * rpt + k * CH, CH)])
        plsc.subcore_barrier()

        def outer(g, _):
            for b in range(IXR):
                c = g * IXR + b

                @pl.when(c + 3 < NCHUNK)
                def _():
                    stage_idx(c + 3, (b + 3) % IXR)

                @pl.when(c >= 1)
                def _():
                    wait_scatter((b + 1) % 2)

                @pl.when(c + 1 < NCHUNK)
                def _():
                    wait_idx((b + 1) % IXR)
                    start_gather(c + 1, (b + 1) % IXR, (b + 1) % 2)

                wait_gather(b % 2)
                start_scatter(c, b % 2)
            return 0

        lax.fori_loop(0, NCHUNK // IXR, outer, 0)
        wait_scatter((NCHUNK - 1) % 2)

        plsc.subcore_barrier()
        for k in range(rpt // CH):
            sl = pl.ds(sid * rpt + k * CH, CH)
            pltpu.sync_copy(acc.at[sl], rows[0])
            pltpu.sync_copy(rows[0], out_hbm.at[cid, sl])

    return body


def _sc_agg(tab, srcp, dst2d, dim):
    mesh = plsc.VectorSubcoreMesh(core_axis_name="c", subcore_axis_name="s")
    f = pl.kernel(
        _make_agg_body(dim),
        out_type=jax.ShapeDtypeStruct((NC, NROW, dim), jnp.float32),
        mesh=mesh,
        compiler_params=pltpu.CompilerParams(
            needs_layout_passes=False, use_tc_tiling_on_sc=False
        ),
        scratch_types=[
            [pltpu.VMEM((CH,), jnp.int32) for _ in range(IXR)],
            pltpu.VMEM((NCHUNK, CH), jnp.int32),
            [pltpu.VMEM((CH, dim), jnp.float32) for _ in range(2)],
            pltpu.VMEM_SHARED((NROW, dim), jnp.float32),
            [pltpu.SemaphoreType.DMA for _ in range(IXR)],
            [pltpu.SemaphoreType.DMA for _ in range(2)],
            [pltpu.SemaphoreType.DMA for _ in range(2)],
        ],
    )
    return f(tab, srcp, dst2d)


def _tc_prep(x_pad, dps, dpd):
    def body(x_ref, dps_ref, dpd_ref, t1_ref, ns_ref, nd_ref):
        ds_ = dps_ref[0] + dps_ref[1]
        dd = dpd_ref[0] + dpd_ref[1]
        ns = jnp.where(ds_ > 0, lax.rsqrt(ds_), 0.0)
        nd = jnp.where(dd > 0, lax.rsqrt(dd), 0.0)
        ns_ref[...] = ns
        nd_ref[...] = nd
        t1_ref[...] = x_ref[...] * ns

    return pl.pallas_call(
        body,
        out_shape=[
            jax.ShapeDtypeStruct((NROW, D_IN), jnp.float32),
            jax.ShapeDtypeStruct((NROW, 1), jnp.float32),
            jax.ShapeDtypeStruct((NROW, 1), jnp.float32),
        ],
    )(x_pad, dps, dpd)


def _tc_mid(p1, ns_col, nd_col, W1, b1r, W2):
    def body(p_ref, ns_ref, nd_ref, w1_ref, b1_ref, w2_ref, t2_ref):
        agg = p_ref[0] + p_ref[1]
        aggn = agg * nd_ref[...]
        h = jnp.dot(aggn, w1_ref[...], preferred_element_type=jnp.float32)
        h = jnp.maximum(h + b1_ref[...], 0.0)
        t2_ref[...] = jnp.dot(
            h * ns_ref[...], w2_ref[...], preferred_element_type=jnp.float32
        )

    return pl.pallas_call(
        body,
        out_shape=jax.ShapeDtypeStruct((NROW, D_OUT), jnp.float32),
    )(p1, ns_col, nd_col, W1, b1r, W2)


def _tc_final(p2, nd_col, b2r):
    def body(p_ref, nd_ref, b2_ref, o_ref):
        o_ref[...] = (p_ref[0, :N] + p_ref[1, :N]) * nd_ref[:N] + b2_ref[...]

    return pl.pallas_call(
        body,
        out_shape=jax.ShapeDtypeStruct((N, D_OUT), jnp.float32),
    )(p2, nd_col, b2r)


def kernel(x, edge_index, W1, b1, W2, b2):
    src = edge_index[0]
    dst = edge_index[1]
    padi = jnp.full((EPAD - E,), N, jnp.int32)
    srcp = jnp.concatenate([src, padi])
    dstp = jnp.concatenate([dst, padi])
    x_pad = jnp.pad(x, ((0, NROW - N), (0, 0)))

    dst2d = dstp.reshape(EPAD // CH, CH)

    degp = _sc_deg(srcp, dstp)
    dps = degp[:, :80, :].reshape(NC, NROW, 1)
    dpd = degp[:, HDST:HDST + 80, :].reshape(NC, NROW, 1)
    t1, ns_col, nd_col = _tc_prep(x_pad, dps, dpd)

    p1 = _sc_agg(t1, srcp, dst2d, D_HID)
    t2 = _tc_mid(p1, ns_col, nd_col, W1, b1.reshape(1, D_HID), W2)

    p2 = _sc_agg(t2, srcp, dst2d, D_OUT)
    return _tc_final(p2, nd_col, b2.reshape(1, D_OUT))


# speedup vs baseline: 5.3622x; 1.0526x over previous
import jax
import jax.numpy as jnp
from jax import lax
from jax.experimental import pallas as pl
from jax.experimental.pallas import tpu as pltpu
from jax.experimental.pallas import tpu_sc as plsc

N = 10000
E = 320000
D_IN = 128
D_HID = 128
D_OUT = 64

NROW = 10240
NC, NS = 2, 16
NW = NC * NS
CH = 128
NCHUNK = 80
EPW = CH * NCHUNK
EPAD = EPW * NW
PAIR = 2 * NCHUNK
CH0 = 112
CH1 = PAIR - CH0
HR = 256
HDST = 128
DEG_RPT = HR // NS


def _deg_body(src_hbm, dst_hbm, out_hbm, sidx, didx, hist, ria, rib, zb, acc):
    cid = lax.axis_index("c")
    sid = lax.axis_index("s")
    wid = sid * NC + cid
    zeros16 = jnp.zeros((16,), jnp.float32)
    ones16 = jnp.ones((16,), jnp.float32)
    iota16 = lax.iota(jnp.int32, 16)

    def zh(r, _):
        for c in range(8):
            hist[r, pl.ds(c * 16, 16)] = zeros16
        return 0

    lax.fori_loop(0, HR, zh, 0)

    for i in range(5):
        ria[pl.ds(i * 16, 16)] = iota16 + i * 16
        rib[pl.ds(i * 16, 16)] = iota16 + (HDST + i * 16)

    for r in range(DEG_RPT):
        for c in range(8):
            zb[r, pl.ds(c * 16, 16)] = zeros16
    pltpu.sync_copy(zb, acc.at[pl.ds(sid * DEG_RPT, DEG_RPT)])

    pltpu.sync_copy(src_hbm.at[pl.ds(wid * EPW, EPW)], sidx)
    pltpu.sync_copy(dst_hbm.at[pl.ds(wid * EPW, EPW)], didx)

    def step(i, _):
        s16 = sidx[pl.ds(i * 16, 16)]
        plsc.addupdate_scatter(
            hist,
            [lax.shift_right_logical(s16, 7), lax.bitwise_and(s16, 127)],
            ones16,
        )
        d16 = didx[pl.ds(i * 16, 16)]
        plsc.addupdate_scatter(
            hist,
            [lax.shift_right_logical(d16, 7) + HDST, lax.bitwise_and(d16, 127)],
            ones16,
        )
        return 0

    lax.fori_loop(0, EPW // 16, step, 0)

    plsc.subcore_barrier()
    pltpu.sync_copy(hist.at[pl.ds(0, 80)], acc.at[ria], add=True)
    pltpu.sync_copy(hist.at[pl.ds(HDST, 80)], acc.at[rib], add=True)
    plsc.subcore_barrier()

    sl = pl.ds(sid * DEG_RPT, DEG_RPT)
    pltpu.sync_copy(acc.at[sl], zb)
    pltpu.sync_copy(zb, out_hbm.at[cid, sl])


def _sc_deg(srcp, dstp):
    mesh = plsc.VectorSubcoreMesh(core_axis_name="c", subcore_axis_name="s")
    f = pl.kernel(
        _deg_body,
        out_type=jax.ShapeDtypeStruct((NC, HR, 128), jnp.float32),
        mesh=mesh,
        compiler_params=pltpu.CompilerParams(needs_layout_passes=False),
        scratch_types=[
            pltpu.VMEM((EPW,), jnp.int32),
            pltpu.VMEM((EPW,), jnp.int32),
            pltpu.VMEM((HR, 128), jnp.float32),
            pltpu.VMEM((80,), jnp.int32),
            pltpu.VMEM((80,), jnp.int32),
            pltpu.VMEM((DEG_RPT, 128), jnp.float32),
            pltpu.VMEM_SHARED((HR, 128), jnp.float32),
        ],
    )
    return f(srcp, dstp)


IXR = 8


def _make_agg_body(dim):
    rpt = NROW // NS

    def body(tab_hbm, src_hbm, dst2d_hbm, out_hbm, sidxb, didx, rows, acc,
             sem_i, sem_g, sem_s):
        cid = lax.axis_index("c")
        sid = lax.axis_index("s")
        zeros16 = jnp.zeros((16,), jnp.float32)
        nchk = jnp.where(cid == 0, CH0, CH1)
        base = sid * PAIR * CH + cid * CH0 * CH

        def stage_idx(c, j):
            pltpu.async_copy(src_hbm.at[pl.ds(base + c * CH, CH)], sidxb[j],
                             sem_i[j])

        def wait_idx(j):
            pltpu.make_async_copy(src_hbm.at[pl.ds(base, CH)], sidxb[j],
                                  sem_i[j]).wait()

        def start_gather(c, j, b):
            pltpu.async_copy(tab_hbm.at[sidxb[j]], rows[b], sem_g[b])

        def wait_gather(b):
            pltpu.make_async_copy(tab_hbm.at[sidxb[0]], rows[b],
                                  sem_g[b]).wait()

        def start_scatter(c, b):
            pltpu.async_copy(rows[b], acc.at[didx.at[c]], sem_s[b], add=True)

        def wait_scatter(b):
            pltpu.make_async_copy(rows[b], acc.at[didx.at[0]], sem_s[b]).wait()

        @pl.when(cid == 0)
        def _():
            pltpu.sync_copy(dst2d_hbm.at[pl.ds(sid * PAIR, CH0)],
                            didx.at[pl.ds(0, CH0)])

        @pl.when(cid == 1)
        def _():
            pltpu.sync_copy(dst2d_hbm.at[pl.ds(sid * PAIR + CH0, CH1)],
                            didx.at[pl.ds(0, CH1)])

        for c in range(3):
            stage_idx(c, c)
        wait_idx(0)
        start_gather(0, 0, 0)

        def zrow(r, _):
            for c in range(dim // 16):
                rows[1][r, pl.ds(c * 16, 16)] = zeros16
            return 0

        lax.fori_loop(0, CH, zrow, 0)
        for k in range(rpt // CH):
            pltpu.sync_copy(rows[1], acc.at[pl.ds(sid * rpt + k * CH, CH)])
        plsc.subcore_barrier()

        def outer(g, _):
            for b in range(IXR):
                c = g * IXR + b

                @pl.when(c + 3 < nchk)
                def _():
                    stage_idx(c + 3, (b + 3) % IXR)

                @pl.when(c >= 1)
                def _():
                    wait_scatter((b + 1) % 2)

                @pl.when(c + 1 < nchk)
                def _():
                    wait_idx((b + 1) % IXR)
                    start_gather(c + 1, (b + 1) % IXR, (b + 1) % 2)

                wait_gather(b % 2)
                start_scatter(c, b % 2)
            return 0

        lax.fori_loop(0, nchk // IXR, outer, 0)
        wait_scatter(1)

        plsc.subcore_barrier()
        for k in range(rpt // CH):
            sl = pl.ds(sid * rpt + k * CH, CH)
            pltpu.sync_copy(acc.at[sl], rows[0])
            pltpu.sync_copy(rows[0], out_hbm.at[cid, sl])

    return body


def _sc_agg(tab, srcp, dst2d, dim):
    mesh = plsc.VectorSubcoreMesh(core_axis_name="c", subcore_axis_name="s")
    f = pl.kernel(
        _make_agg_body(dim),
        out_type=jax.ShapeDtypeStruct((NC, NROW, dim), jnp.float32),
        mesh=mesh,
        compiler_params=pltpu.CompilerParams(
            needs_layout_passes=False, use_tc_tiling_on_sc=False
        ),
        scratch_types=[
            [pltpu.VMEM((CH,), jnp.int32) for _ in range(IXR)],
            pltpu.VMEM((CH0, CH), jnp.int32),
            [pltpu.VMEM((CH, dim), jnp.float32) for _ in range(2)],
            pltpu.VMEM_SHARED((NROW, dim), jnp.float32),
            [pltpu.SemaphoreType.DMA for _ in range(IXR)],
            [pltpu.SemaphoreType.DMA for _ in range(2)],
            [pltpu.SemaphoreType.DMA for _ in range(2)],
        ],
    )
    return f(tab, srcp, dst2d)


def _tc_prep(x_pad, dps, dpd):
    def body(x_ref, dps_ref, dpd_ref, t1_ref, ns_ref, nd_ref):
        ds_ = dps_ref[0] + dps_ref[1]
        dd = dpd_ref[0] + dpd_ref[1]
        ns = jnp.where(ds_ > 0, lax.rsqrt(ds_), 0.0)
        nd = jnp.where(dd > 0, lax.rsqrt(dd), 0.0)
        ns_ref[...] = ns
        nd_ref[...] = nd
        t1_ref[...] = x_ref[...] * ns

    return pl.pallas_call(
        body,
        out_shape=[
            jax.ShapeDtypeStruct((NROW, D_IN), jnp.float32),
            jax.ShapeDtypeStruct((NROW, 1), jnp.float32),
            jax.ShapeDtypeStruct((NROW, 1), jnp.float32),
        ],
    )(x_pad, dps, dpd)


def _tc_mid(p1, ns_col, nd_col, W1, b1r, W2):
    def body(p_ref, ns_ref, nd_ref, w1_ref, b1_ref, w2_ref, t2_ref):
        agg = p_ref[0] + p_ref[1]
        aggn = agg * nd_ref[...]
        h = jnp.dot(aggn, w1_ref[...], preferred_element_type=jnp.float32)
        h = jnp.maximum(h + b1_ref[...], 0.0)
        t2_ref[...] = jnp.dot(
            h * ns_ref[...], w2_ref[...], preferred_element_type=jnp.float32
        )

    return pl.pallas_call(
        body,
        out_shape=jax.ShapeDtypeStruct((NROW, D_OUT), jnp.float32),
    )(p1, ns_col, nd_col, W1, b1r, W2)


def _tc_final(p2, nd_col, b2r):
    def body(p_ref, nd_ref, b2_ref, o_ref):
        o_ref[...] = (p_ref[0, :N] + p_ref[1, :N]) * nd_ref[:N] + b2_ref[...]

    return pl.pallas_call(
        body,
        out_shape=jax.ShapeDtypeStruct((N, D_OUT), jnp.float32),
    )(p2, nd_col, b2r)


def kernel(x, edge_index, W1, b1, W2, b2):
    src = edge_index[0]
    dst = edge_index[1]
    padi = jnp.full((EPAD - E,), N, jnp.int32)
    srcp = jnp.concatenate([src, padi])
    dstp = jnp.concatenate([dst, padi])
    x_pad = jnp.pad(x, ((0, NROW - N), (0, 0)))

    dst2d = dstp.reshape(EPAD // CH, CH)

    degp = _sc_deg(srcp, dstp)
    dps = degp[:, :80, :].reshape(NC, NROW, 1)
    dpd = degp[:, HDST:HDST + 80, :].reshape(NC, NROW, 1)
    t1, ns_col, nd_col = _tc_prep(x_pad, dps, dpd)

    p1 = _sc_agg(t1, srcp, dst2d, D_HID)
    t2 = _tc_mid(p1, ns_col, nd_col, W1, b1.reshape(1, D_HID), W2)

    p2 = _sc_agg(t2, srcp, dst2d, D_OUT)
    return _tc_final(p2, nd_col, b2.reshape(1, D_OUT))


# speedup vs baseline: 9.4245x; 1.7576x over previous
import jax
import jax.numpy as jnp
from jax import lax
from jax.experimental import pallas as pl
from jax.experimental.pallas import tpu as pltpu
from jax.experimental.pallas import tpu_sc as plsc

N = 10000
E = 320000
D_IN = 128
D_HID = 128
D_OUT = 64

NROW = 10240
NC, NS = 2, 16
NW = NC * NS
CH = 128
NCHUNK = 80
EPW = CH * NCHUNK
EPAD = EPW * NW
PAIR = 2 * NCHUNK
CH0 = 112
CH1 = PAIR - CH0
HR = 256
HDST = 128
DEG_RPT = HR // NS


def _deg_body(src_hbm, dst_hbm, out_hbm, sidx, didx, hist, ria, rib, zb, acc):
    cid = lax.axis_index("c")
    sid = lax.axis_index("s")
    wid = sid * NC + cid
    zeros16 = jnp.zeros((16,), jnp.float32)
    ones16 = jnp.ones((16,), jnp.float32)
    iota16 = lax.iota(jnp.int32, 16)

    def zh(r, _):
        for c in range(8):
            hist[r, pl.ds(c * 16, 16)] = zeros16
        return 0

    lax.fori_loop(0, HR, zh, 0)

    for i in range(5):
        ria[pl.ds(i * 16, 16)] = iota16 + i * 16
        rib[pl.ds(i * 16, 16)] = iota16 + (HDST + i * 16)

    for r in range(DEG_RPT):
        for c in range(8):
            zb[r, pl.ds(c * 16, 16)] = zeros16
    pltpu.sync_copy(zb, acc.at[pl.ds(sid * DEG_RPT, DEG_RPT)])

    pltpu.sync_copy(src_hbm.at[pl.ds(wid * EPW, EPW)], sidx)
    pltpu.sync_copy(dst_hbm.at[pl.ds(wid * EPW, EPW)], didx)

    def step(i, _):
        s16 = sidx[pl.ds(i * 16, 16)]
        plsc.addupdate_scatter(
            hist,
            [lax.shift_right_logical(s16, 7), lax.bitwise_and(s16, 127)],
            ones16,
        )
        d16 = didx[pl.ds(i * 16, 16)]
        plsc.addupdate_scatter(
            hist,
            [lax.shift_right_logical(d16, 7) + HDST, lax.bitwise_and(d16, 127)],
            ones16,
        )
        return 0

    lax.fori_loop(0, EPW // 16, step, 0)

    plsc.subcore_barrier()
    pltpu.sync_copy(hist.at[pl.ds(0, 80)], acc.at[ria], add=True)
    pltpu.sync_copy(hist.at[pl.ds(HDST, 80)], acc.at[rib], add=True)
    plsc.subcore_barrier()

    sl = pl.ds(sid * DEG_RPT, DEG_RPT)
    pltpu.sync_copy(acc.at[sl], zb)
    pltpu.sync_copy(zb, out_hbm.at[cid, sl])


def _sc_deg(srcp, dstp):
    mesh = plsc.VectorSubcoreMesh(core_axis_name="c", subcore_axis_name="s")
    f = pl.kernel(
        _deg_body,
        out_type=jax.ShapeDtypeStruct((NC, HR, 128), jnp.float32),
        mesh=mesh,
        compiler_params=pltpu.CompilerParams(needs_layout_passes=False),
        scratch_types=[
            pltpu.VMEM((EPW,), jnp.int32),
            pltpu.VMEM((EPW,), jnp.int32),
            pltpu.VMEM((HR, 128), jnp.float32),
            pltpu.VMEM((80,), jnp.int32),
            pltpu.VMEM((80,), jnp.int32),
            pltpu.VMEM((DEG_RPT, 128), jnp.float32),
            pltpu.VMEM_SHARED((HR, 128), jnp.float32),
        ],
    )
    return f(srcp, dstp)


IXR = 8
DH = 64


def _make_sh_agg_body(nchk, half_edges):
    rpt = NROW // NS

    def body(tab_hbm, src_hbm, dst2d_hbm, out_hbm, sidxb, didx, rows,
             tab_sh, acc, sem_i, sem_g, sem_s):
        cid = lax.axis_index("c")
        sid = lax.axis_index("s")
        zeros16 = jnp.zeros((16,), jnp.float32)
        wix = sid * NC + cid if half_edges else sid
        base = wix * nchk * CH

        def stage_idx(c, j):
            pltpu.async_copy(src_hbm.at[pl.ds(base + c * CH, CH)], sidxb[j],
                             sem_i[j])

        def wait_idx(j):
            pltpu.make_async_copy(src_hbm.at[pl.ds(base, CH)], sidxb[j],
                                  sem_i[j]).wait()

        def start_gather(j, b):
            pltpu.async_copy(tab_sh.at[sidxb[j]], rows[b], sem_g[b])

        def wait_gather(b):
            pltpu.make_async_copy(tab_sh.at[sidxb[0]], rows[b],
                                  sem_g[b]).wait()

        def start_scatter(c, b):
            pltpu.async_copy(rows[b], acc.at[didx.at[c]], sem_s[b], add=True)

        def wait_scatter(b):
            pltpu.make_async_copy(rows[b], acc.at[didx.at[0]], sem_s[b]).wait()

        sl0 = pl.ds(sid * rpt, rpt)
        pltpu.sync_copy(tab_hbm.at[cid].at[sl0], tab_sh.at[sl0])
        pltpu.sync_copy(dst2d_hbm.at[pl.ds(wix * nchk, nchk)], didx)
        for c in range(3):
            stage_idx(c, c)

        def zrow(r, _):
            for c in range(DH // 16):
                rows[1][r, pl.ds(c * 16, 16)] = zeros16
            return 0

        lax.fori_loop(0, CH, zrow, 0)
        for k in range(rpt // CH):
            pltpu.sync_copy(rows[1], acc.at[pl.ds(sid * rpt + k * CH, CH)])
        plsc.subcore_barrier()

        wait_idx(0)
        start_gather(0, 0)

        def outer(g, _):
            for b in range(IXR):
                c = g * IXR + b

                @pl.when(c + 3 < nchk)
                def _():
                    stage_idx(c + 3, (b + 3) % IXR)

                @pl.when(c >= 1)
                def _():
                    wait_scatter((b + 1) % 2)

                @pl.when(c + 1 < nchk)
                def _():
                    wait_idx((b + 1) % IXR)
                    start_gather((b + 1) % IXR, (b + 1) % 2)

                wait_gather(b % 2)
                start_scatter(c, b % 2)
            return 0

        lax.fori_loop(0, nchk // IXR, outer, 0)
        wait_scatter((nchk - 1) % 2)

        plsc.subcore_barrier()
        for k in range(rpt // CH):
            sl = pl.ds(sid * rpt + k * CH, CH)
            pltpu.sync_copy(acc.at[sl], rows[0])
            pltpu.sync_copy(rows[0], out_hbm.at[cid, sl])

    return body


def _sc_agg(tab2, srcp, dst2d, half_edges):
    nchk = NCHUNK if half_edges else 2 * NCHUNK
    mesh = plsc.VectorSubcoreMesh(core_axis_name="c", subcore_axis_name="s")
    f = pl.kernel(
        _make_sh_agg_body(nchk, half_edges),
        out_type=jax.ShapeDtypeStruct((NC, NROW, DH), jnp.float32),
        mesh=mesh,
        compiler_params=pltpu.CompilerParams(
            needs_layout_passes=False, use_tc_tiling_on_sc=False
        ),
        scratch_types=[
            [pltpu.VMEM((CH,), jnp.int32) for _ in range(IXR)],
            pltpu.VMEM((nchk, CH), jnp.int32),
            [pltpu.VMEM((CH, DH), jnp.float32) for _ in range(2)],
            pltpu.VMEM_SHARED((NROW, DH), jnp.float32),
            pltpu.VMEM_SHARED((NROW, DH), jnp.float32),
            [pltpu.SemaphoreType.DMA for _ in range(IXR)],
            [pltpu.SemaphoreType.DMA for _ in range(2)],
            [pltpu.SemaphoreType.DMA for _ in range(2)],
        ],
    )
    return f(tab2, srcp, dst2d)


def _tc_prep(x_pad, dps, dpd):
    def body(x_ref, dps_ref, dpd_ref, t1_ref, ns_ref, nd_ref):
        ds_ = dps_ref[0] + dps_ref[1]
        dd = dpd_ref[0] + dpd_ref[1]
        ns = jnp.where(ds_ > 0, lax.rsqrt(ds_), 0.0)
        nd = jnp.where(dd > 0, lax.rsqrt(dd), 0.0)
        ns_ref[...] = ns
        nd_ref[...] = nd
        t1 = x_ref[...] * ns
        t1_ref[0] = t1[:, :DH]
        t1_ref[1] = t1[:, DH:]

    return pl.pallas_call(
        body,
        out_shape=[
            jax.ShapeDtypeStruct((NC, NROW, DH), jnp.float32),
            jax.ShapeDtypeStruct((NROW, 1), jnp.float32),
            jax.ShapeDtypeStruct((NROW, 1), jnp.float32),
        ],
    )(x_pad, dps, dpd)


def _tc_mid(p1, ns_col, nd_col, W1, b1r, W2):
    def body(p_ref, ns_ref, nd_ref, w1_ref, b1_ref, w2_ref, t2_ref):
        nd = nd_ref[...]
        h = jnp.dot(p_ref[0] * nd, w1_ref[:DH],
                    preferred_element_type=jnp.float32)
        h = h + jnp.dot(p_ref[1] * nd, w1_ref[DH:],
                        preferred_element_type=jnp.float32)
        h = jnp.maximum(h + b1_ref[...], 0.0)
        t2 = jnp.dot(
            h * ns_ref[...], w2_ref[...], preferred_element_type=jnp.float32
        )
        t2_ref[0] = t2
        t2_ref[1] = t2

    return pl.pallas_call(
        body,
        out_shape=jax.ShapeDtypeStruct((NC, NROW, D_OUT), jnp.float32),
    )(p1, ns_col, nd_col, W1, b1r, W2)


def _tc_final(p2, nd_col, b2r):
    def body(p_ref, nd_ref, b2_ref, o_ref):
        o_ref[...] = (p_ref[0, :N] + p_ref[1, :N]) * nd_ref[:N] + b2_ref[...]

    return pl.pallas_call(
        body,
        out_shape=jax.ShapeDtypeStruct((N, D_OUT), jnp.float32),
    )(p2, nd_col, b2r)


def kernel(x, edge_index, W1, b1, W2, b2):
    src = edge_index[0]
    dst = edge_index[1]
    padi = jnp.full((EPAD - E,), N, jnp.int32)
    srcp = jnp.concatenate([src, padi])
    dstp = jnp.concatenate([dst, padi])
    x_pad = jnp.pad(x, ((0, NROW - N), (0, 0)))

    dst2d = dstp.reshape(EPAD // CH, CH)

    degp = _sc_deg(srcp, dstp)
    dps = degp[:, :80, :].reshape(NC, NROW, 1)
    dpd = degp[:, HDST:HDST + 80, :].reshape(NC, NROW, 1)
    t1h, ns_col, nd_col = _tc_prep(x_pad, dps, dpd)

    p1 = _sc_agg(t1h, srcp, dst2d, False)
    t2s = _tc_mid(p1, ns_col, nd_col, W1, b1.reshape(1, D_HID), W2)

    p2 = _sc_agg(t2s, srcp, dst2d, True)
    return _tc_final(p2, nd_col, b2.reshape(1, D_OUT))


# speedup vs baseline: 10.5565x; 1.1201x over previous
import jax
import jax.numpy as jnp
from jax import lax
from jax.experimental import pallas as pl
from jax.experimental.pallas import tpu as pltpu
from jax.experimental.pallas import tpu_sc as plsc

N = 10000
E = 320000
D_IN = 128
D_HID = 128
D_OUT = 64

NROW = 10240
NC, NS = 2, 16
NW = NC * NS
CH = 128
NCHUNK = 80
EPW = CH * NCHUNK
EPAD = EPW * NW
PAIR = 2 * NCHUNK
CH0 = 112
CH1 = PAIR - CH0
HR = 256
HDST = 128
DEG_RPT = HR // NS


def _deg_body(src_hbm, dst_hbm, out_hbm, sidx, didx, hist, ria, rib, zb, acc):
    cid = lax.axis_index("c")
    sid = lax.axis_index("s")
    wid = sid * NC + cid
    zeros16 = jnp.zeros((16,), jnp.float32)
    ones16 = jnp.ones((16,), jnp.float32)
    iota16 = lax.iota(jnp.int32, 16)

    def zh(r, _):
        for c in range(8):
            hist[r, pl.ds(c * 16, 16)] = zeros16
        return 0

    lax.fori_loop(0, HR, zh, 0)

    for i in range(5):
        ria[pl.ds(i * 16, 16)] = iota16 + i * 16
        rib[pl.ds(i * 16, 16)] = iota16 + (HDST + i * 16)

    for r in range(DEG_RPT):
        for c in range(8):
            zb[r, pl.ds(c * 16, 16)] = zeros16
    pltpu.sync_copy(zb, acc.at[pl.ds(sid * DEG_RPT, DEG_RPT)])

    pltpu.sync_copy(src_hbm.at[pl.ds(wid * EPW, EPW)], sidx)
    pltpu.sync_copy(dst_hbm.at[pl.ds(wid * EPW, EPW)], didx)

    def step(i, _):
        s16 = sidx[pl.ds(i * 16, 16)]
        plsc.addupdate_scatter(
            hist,
            [lax.shift_right_logical(s16, 7), lax.bitwise_and(s16, 127)],
            ones16,
        )
        d16 = didx[pl.ds(i * 16, 16)]
        plsc.addupdate_scatter(
            hist,
            [lax.shift_right_logical(d16, 7) + HDST, lax.bitwise_and(d16, 127)],
            ones16,
        )
        return 0

    lax.fori_loop(0, EPW // 16, step, 0)

    plsc.subcore_barrier()
    pltpu.sync_copy(hist.at[pl.ds(0, 80)], acc.at[ria], add=True)
    pltpu.sync_copy(hist.at[pl.ds(HDST, 80)], acc.at[rib], add=True)
    plsc.subcore_barrier()

    sl = pl.ds(sid * DEG_RPT, DEG_RPT)
    pltpu.sync_copy(acc.at[sl], zb)
    pltpu.sync_copy(zb, out_hbm.at[cid, sl])


def _sc_deg(srcp, dstp):
    mesh = plsc.VectorSubcoreMesh(core_axis_name="c", subcore_axis_name="s")
    f = pl.kernel(
        _deg_body,
        out_type=jax.ShapeDtypeStruct((NC, HR, 128), jnp.float32),
        mesh=mesh,
        compiler_params=pltpu.CompilerParams(needs_layout_passes=False),
        scratch_types=[
            pltpu.VMEM((EPW,), jnp.int32),
            pltpu.VMEM((EPW,), jnp.int32),
            pltpu.VMEM((HR, 128), jnp.float32),
            pltpu.VMEM((80,), jnp.int32),
            pltpu.VMEM((80,), jnp.int32),
            pltpu.VMEM((DEG_RPT, 128), jnp.float32),
            pltpu.VMEM_SHARED((HR, 128), jnp.float32),
        ],
    )
    return f(srcp, dstp)


IXR = 8
DH = 64


def _make_sh_agg_body(nchk, half_edges):
    rpt = NROW // NS

    def body(tab_hbm, src_hbm, dst3d_hbm, out_hbm, sidxb, didxb, rows,
             tab_sh, acc, sem_i, sem_g, sem_s):
        cid = lax.axis_index("c")
        sid = lax.axis_index("s")
        zeros16 = jnp.zeros((16,), jnp.float32)
        wix = sid * NC + cid if half_edges else sid
        base = wix * nchk * CH

        def stage_idx(c, j):
            pltpu.async_copy(src_hbm.at[pl.ds(base + c * CH, CH)], sidxb[j],
                             sem_i[j])
            pltpu.async_copy(dst3d_hbm.at[wix * nchk + c], didxb[j],
                             sem_i[j])

        def wait_idx(j):
            pltpu.make_async_copy(src_hbm.at[pl.ds(base, CH)], sidxb[j],
                                  sem_i[j]).wait()
            pltpu.make_async_copy(dst3d_hbm.at[0], didxb[j],
                                  sem_i[j]).wait()

        def start_gather(j, b):
            pltpu.async_copy(tab_sh.at[sidxb[j]], rows[b], sem_g[b])

        def wait_gather(b):
            pltpu.make_async_copy(tab_sh.at[sidxb[0]], rows[b],
                                  sem_g[b]).wait()

        def start_scatter(j, b):
            pltpu.async_copy(rows[b], acc.at[didxb[j].at[0]], sem_s[b],
                             add=True)

        def wait_scatter(b):
            pltpu.make_async_copy(rows[b], acc.at[didxb[0].at[0]],
                                  sem_s[b]).wait()

        sl0 = pl.ds(sid * rpt, rpt)
        pltpu.sync_copy(tab_hbm.at[cid].at[sl0], tab_sh.at[sl0])
        for c in range(4):
            stage_idx(c, c)

        def zrow(r, _):
            for c in range(DH // 16):
                rows[1][r, pl.ds(c * 16, 16)] = zeros16
            return 0

        lax.fori_loop(0, CH, zrow, 0)
        for k in range(rpt // CH):
            pltpu.sync_copy(rows[1], acc.at[pl.ds(sid * rpt + k * CH, CH)])
        plsc.subcore_barrier()

        wait_idx(0)
        start_gather(0, 0)
        wait_idx(1)
        start_gather(1, 1)

        def outer(g, _):
            for b in range(IXR):
                c = g * IXR + b

                @pl.when(c + 4 < nchk)
                def _():
                    stage_idx(c + 4, (b + 4) % IXR)

                @pl.when(c >= 2)
                def _():
                    wait_scatter((b + 2) % 4)

                @pl.when(c + 2 < nchk)
                def _():
                    wait_idx((b + 2) % IXR)
                    start_gather((b + 2) % IXR, (b + 2) % 4)

                wait_gather(b % 4)
                start_scatter(b % IXR, b % 4)
            return 0

        lax.fori_loop(0, nchk // IXR, outer, 0)
        wait_scatter((nchk - 2) % 4)
        wait_scatter((nchk - 1) % 4)

        plsc.subcore_barrier()
        for k in range(rpt // CH):
            sl = pl.ds(sid * rpt + k * CH, CH)
            pltpu.sync_copy(acc.at[sl], rows[0])
            pltpu.sync_copy(rows[0], out_hbm.at[cid, sl])

    return body


def _sc_agg(tab2, srcp, dst3d, half_edges):
    nchk = NCHUNK if half_edges else 2 * NCHUNK
    mesh = plsc.VectorSubcoreMesh(core_axis_name="c", subcore_axis_name="s")
    f = pl.kernel(
        _make_sh_agg_body(nchk, half_edges),
        out_type=jax.ShapeDtypeStruct((NC, NROW, DH), jnp.float32),
        mesh=mesh,
        compiler_params=pltpu.CompilerParams(
            needs_layout_passes=False, use_tc_tiling_on_sc=False
        ),
        scratch_types=[
            [pltpu.VMEM((CH,), jnp.int32) for _ in range(IXR)],
            [pltpu.VMEM((1, CH), jnp.int32) for _ in range(IXR)],
            [pltpu.VMEM((CH, DH), jnp.float32) for _ in range(4)],
            pltpu.VMEM_SHARED((NROW, DH), jnp.float32),
            pltpu.VMEM_SHARED((NROW, DH), jnp.float32),
            [pltpu.SemaphoreType.DMA for _ in range(IXR)],
            [pltpu.SemaphoreType.DMA for _ in range(4)],
            [pltpu.SemaphoreType.DMA for _ in range(4)],
        ],
    )
    return f(tab2, srcp, dst3d)


def _tc_prep(x_pad, dps, dpd):
    def body(x_ref, dps_ref, dpd_ref, t1_ref, ns_ref, nd_ref):
        ds_ = dps_ref[0] + dps_ref[1]
        dd = dpd_ref[0] + dpd_ref[1]
        ns = jnp.where(ds_ > 0, lax.rsqrt(ds_), 0.0)
        nd = jnp.where(dd > 0, lax.rsqrt(dd), 0.0)
        ns_ref[...] = ns
        nd_ref[...] = nd
        t1 = x_ref[...] * ns
        t1_ref[0] = t1[:, :DH]
        t1_ref[1] = t1[:, DH:]

    return pl.pallas_call(
        body,
        out_shape=[
            jax.ShapeDtypeStruct((NC, NROW, DH), jnp.float32),
            jax.ShapeDtypeStruct((NROW, 1), jnp.float32),
            jax.ShapeDtypeStruct((NROW, 1), jnp.float32),
        ],
    )(x_pad, dps, dpd)


def _tc_mid(p1, ns_col, nd_col, W1, b1r, W2):
    def body(p_ref, ns_ref, nd_ref, w1_ref, b1_ref, w2_ref, t2_ref):
        nd = nd_ref[...]
        h = jnp.dot(p_ref[0] * nd, w1_ref[:DH],
                    preferred_element_type=jnp.float32)
        h = h + jnp.dot(p_ref[1] * nd, w1_ref[DH:],
                        preferred_element_type=jnp.float32)
        h = jnp.maximum(h + b1_ref[...], 0.0)
        t2 = jnp.dot(
            h * ns_ref[...], w2_ref[...], preferred_element_type=jnp.float32
        )
        t2_ref[0] = t2
        t2_ref[1] = t2

    return pl.pallas_call(
        body,
        out_shape=jax.ShapeDtypeStruct((NC, NROW, D_OUT), jnp.float32),
    )(p1, ns_col, nd_col, W1, b1r, W2)


def _tc_final(p2, nd_col, b2r):
    def body(p_ref, nd_ref, b2_ref, o_ref):
        o_ref[...] = (p_ref[0, :N] + p_ref[1, :N]) * nd_ref[:N] + b2_ref[...]

    return pl.pallas_call(
        body,
        out_shape=jax.ShapeDtypeStruct((N, D_OUT), jnp.float32),
    )(p2, nd_col, b2r)


def kernel(x, edge_index, W1, b1, W2, b2):
    src = edge_index[0]
    dst = edge_index[1]
    padi = jnp.full((EPAD - E,), N, jnp.int32)
    srcp = jnp.concatenate([src, padi])
    dstp = jnp.concatenate([dst, padi])
    x_pad = jnp.pad(x, ((0, NROW - N), (0, 0)))

    dst2d = dstp.reshape(EPAD // CH, CH)
    dst3d = dstp.reshape(EPAD // CH, 1, CH)

    degp = _sc_deg(srcp, dstp)
    dps = degp[:, :80, :].reshape(NC, NROW, 1)
    dpd = degp[:, HDST:HDST + 80, :].reshape(NC, NROW, 1)
    t1h, ns_col, nd_col = _tc_prep(x_pad, dps, dpd)

    p1 = _sc_agg(t1h, srcp, dst3d, False)
    t2s = _tc_mid(p1, ns_col, nd_col, W1, b1.reshape(1, D_HID), W2)

    p2 = _sc_agg(t2s, srcp, dst3d, True)
    return _tc_final(p2, nd_col, b2.reshape(1, D_OUT))


# speedup vs baseline: 10.6777x; 1.0115x over previous
import jax
import jax.numpy as jnp
from jax import lax
from jax.experimental import pallas as pl
from jax.experimental.pallas import tpu as pltpu
from jax.experimental.pallas import tpu_sc as plsc

N = 10000
E = 320000
D_IN = 128
D_HID = 128
D_OUT = 64

NROW = 10240
NC, NS = 2, 16
NW = NC * NS
CH = 128
NCHUNK = 80
EPW = CH * NCHUNK
EPAD = EPW * NW
PAIR = 2 * NCHUNK
CH0 = 112
CH1 = PAIR - CH0
HR = 256
HDST = 128
DEG_RPT = HR // NS


DEG0 = 13184
DEG1 = 2 * EPW - DEG0


def _deg_body(src_hbm, dst_hbm, out_hbm, sidx, didx, hist, ria, rib, zb, acc):
    cid = lax.axis_index("c")
    sid = lax.axis_index("s")
    zeros16 = jnp.zeros((16,), jnp.float32)
    ones16 = jnp.ones((16,), jnp.float32)
    iota16 = lax.iota(jnp.int32, 16)
    nedge = jnp.where(cid == 0, DEG0, DEG1)
    base = sid * 2 * EPW + cid * DEG0

    def zh(r, _):
        for c in range(8):
            hist[r, pl.ds(c * 16, 16)] = zeros16
        return 0

    lax.fori_loop(0, HR, zh, 0)

    for i in range(5):
        ria[pl.ds(i * 16, 16)] = iota16 + i * 16
        rib[pl.ds(i * 16, 16)] = iota16 + (HDST + i * 16)

    for r in range(DEG_RPT):
        for c in range(8):
            zb[r, pl.ds(c * 16, 16)] = zeros16
    pltpu.sync_copy(zb, acc.at[pl.ds(sid * DEG_RPT, DEG_RPT)])

    @pl.when(cid == 0)
    def _():
        pltpu.sync_copy(src_hbm.at[pl.ds(base, DEG0)], sidx.at[pl.ds(0, DEG0)])
        pltpu.sync_copy(dst_hbm.at[pl.ds(base, DEG0)], didx.at[pl.ds(0, DEG0)])

    @pl.when(cid == 1)
    def _():
        pltpu.sync_copy(src_hbm.at[pl.ds(base, DEG1)], sidx.at[pl.ds(0, DEG1)])
        pltpu.sync_copy(dst_hbm.at[pl.ds(base, DEG1)], didx.at[pl.ds(0, DEG1)])

    def step(i, _):
        s16 = sidx[pl.ds(i * 16, 16)]
        plsc.addupdate_scatter(
            hist,
            [lax.shift_right_logical(s16, 7), lax.bitwise_and(s16, 127)],
            ones16,
        )
        d16 = didx[pl.ds(i * 16, 16)]
        plsc.addupdate_scatter(
            hist,
            [lax.shift_right_logical(d16, 7) + HDST, lax.bitwise_and(d16, 127)],
            ones16,
        )
        return 0

    lax.fori_loop(0, nedge // 16, step, 0)

    plsc.subcore_barrier()
    pltpu.sync_copy(hist.at[pl.ds(0, 80)], acc.at[ria], add=True)
    pltpu.sync_copy(hist.at[pl.ds(HDST, 80)], acc.at[rib], add=True)
    plsc.subcore_barrier()

    sl = pl.ds(sid * DEG_RPT, DEG_RPT)
    pltpu.sync_copy(acc.at[sl], zb)
    pltpu.sync_copy(zb, out_hbm.at[cid, sl])


def _sc_deg(srcp, dstp):
    mesh = plsc.VectorSubcoreMesh(core_axis_name="c", subcore_axis_name="s")
    f = pl.kernel(
        _deg_body,
        out_type=jax.ShapeDtypeStruct((NC, HR, 128), jnp.float32),
        mesh=mesh,
        compiler_params=pltpu.CompilerParams(needs_layout_passes=False),
        scratch_types=[
            pltpu.VMEM((DEG0,), jnp.int32),
            pltpu.VMEM((DEG0,), jnp.int32),
            pltpu.VMEM((HR, 128), jnp.float32),
            pltpu.VMEM((80,), jnp.int32),
            pltpu.VMEM((80,), jnp.int32),
            pltpu.VMEM((DEG_RPT, 128), jnp.float32),
            pltpu.VMEM_SHARED((HR, 128), jnp.float32),
        ],
    )
    return f(srcp, dstp)


IXR = 8
DH = 64


def _make_sh_agg_body(nchk, half_edges):
    rpt = NROW // NS

    def body(tab_hbm, src_hbm, dst3d_hbm, out_hbm, sidxb, didxb, rows,
             tab_sh, acc, sem_i, sem_g, sem_s):
        cid = lax.axis_index("c")
        sid = lax.axis_index("s")
        zeros16 = jnp.zeros((16,), jnp.float32)
        wix = sid * NC + cid if half_edges else sid
        base = wix * nchk * CH

        def stage_idx(c, j):
            pltpu.async_copy(src_hbm.at[pl.ds(base + c * CH, CH)], sidxb[j],
                             sem_i[j])
            pltpu.async_copy(dst3d_hbm.at[wix * nchk + c], didxb[j],
                             sem_i[j])

        def wait_idx(j):
            pltpu.make_async_copy(src_hbm.at[pl.ds(base, CH)], sidxb[j],
                                  sem_i[j]).wait()
            pltpu.make_async_copy(dst3d_hbm.at[0], didxb[j],
                                  sem_i[j]).wait()

        def start_gather(j, b):
            pltpu.async_copy(tab_sh.at[sidxb[j]], rows[b], sem_g[b])

        def wait_gather(b):
            pltpu.make_async_copy(tab_sh.at[sidxb[0]], rows[b],
                                  sem_g[b]).wait()

        def start_scatter(j, b):
            pltpu.async_copy(rows[b], acc.at[didxb[j].at[0]], sem_s[b],
                             add=True)

        def wait_scatter(b):
            pltpu.make_async_copy(rows[b], acc.at[didxb[0].at[0]],
                                  sem_s[b]).wait()

        sl0 = pl.ds(sid * rpt, rpt)
        pltpu.sync_copy(tab_hbm.at[cid].at[sl0], tab_sh.at[sl0])
        for c in range(4):
            stage_idx(c, c)

        def zrow(r, _):
            for c in range(DH // 16):
                rows[1][r, pl.ds(c * 16, 16)] = zeros16
            return 0

        lax.fori_loop(0, CH, zrow, 0)
        for k in range(rpt // CH):
            pltpu.sync_copy(rows[1], acc.at[pl.ds(sid * rpt + k * CH, CH)])
        plsc.subcore_barrier()

        wait_idx(0)
        start_gather(0, 0)
        wait_idx(1)
        start_gather(1, 1)

        def outer(g, _):
            for b in range(IXR):
                c = g * IXR + b

                @pl.when(c + 4 < nchk)
                def _():
                    stage_idx(c + 4, (b + 4) % IXR)

                @pl.when(c >= 2)
                def _():
                    wait_scatter((b + 2) % 4)

                @pl.when(c + 2 < nchk)
                def _():
                    wait_idx((b + 2) % IXR)
                    start_gather((b + 2) % IXR, (b + 2) % 4)

                wait_gather(b % 4)
                start_scatter(b % IXR, b % 4)
            return 0

        lax.fori_loop(0, nchk // IXR, outer, 0)
        wait_scatter((nchk - 2) % 4)
        wait_scatter((nchk - 1) % 4)

        plsc.subcore_barrier()
        for k in range(rpt // CH):
            sl = pl.ds(sid * rpt + k * CH, CH)
            pltpu.sync_copy(acc.at[sl], rows[0])
            pltpu.sync_copy(rows[0], out_hbm.at[cid, sl])

    return body


def _sc_agg(tab2, srcp, dst3d, half_edges):
    nchk = NCHUNK if half_edges else 2 * NCHUNK
    mesh = plsc.VectorSubcoreMesh(core_axis_name="c", subcore_axis_name="s")
    f = pl.kernel(
        _make_sh_agg_body(nchk, half_edges),
        out_type=jax.ShapeDtypeStruct((NC, NROW, DH), jnp.float32),
        mesh=mesh,
        compiler_params=pltpu.CompilerParams(
            needs_layout_passes=False, use_tc_tiling_on_sc=False
        ),
        scratch_types=[
            [pltpu.VMEM((CH,), jnp.int32) for _ in range(IXR)],
            [pltpu.VMEM((1, CH), jnp.int32) for _ in range(IXR)],
            [pltpu.VMEM((CH, DH), jnp.float32) for _ in range(4)],
            pltpu.VMEM_SHARED((NROW, DH), jnp.float32),
            pltpu.VMEM_SHARED((NROW, DH), jnp.float32),
            [pltpu.SemaphoreType.DMA for _ in range(IXR)],
            [pltpu.SemaphoreType.DMA for _ in range(4)],
            [pltpu.SemaphoreType.DMA for _ in range(4)],
        ],
    )
    return f(tab2, srcp, dst3d)


def _tc_prep(x_pad, dps, dpd):
    def body(x_ref, dps_ref, dpd_ref, t1_ref, ns_ref, nd_ref):
        ds_ = dps_ref[0] + dps_ref[1]
        dd = dpd_ref[0] + dpd_ref[1]
        ns = jnp.where(ds_ > 0, lax.rsqrt(ds_), 0.0)
        nd = jnp.where(dd > 0, lax.rsqrt(dd), 0.0)
        ns_ref[...] = ns
        nd_ref[...] = nd
        t1 = x_ref[...] * ns
        t1_ref[0] = t1[:, :DH]
        t1_ref[1] = t1[:, DH:]

    return pl.pallas_call(
        body,
        out_shape=[
            jax.ShapeDtypeStruct((NC, NROW, DH), jnp.float32),
            jax.ShapeDtypeStruct((NROW, 1), jnp.float32),
            jax.ShapeDtypeStruct((NROW, 1), jnp.float32),
        ],
    )(x_pad, dps, dpd)


def _tc_mid(p1, ns_col, nd_col, W1, b1r, W2):
    def body(p_ref, ns_ref, nd_ref, w1_ref, b1_ref, w2_ref, t2_ref):
        nd = nd_ref[...]
        h = jnp.dot(p_ref[0] * nd, w1_ref[:DH],
                    preferred_element_type=jnp.float32)
        h = h + jnp.dot(p_ref[1] * nd, w1_ref[DH:],
                        preferred_element_type=jnp.float32)
        h = jnp.maximum(h + b1_ref[...], 0.0)
        t2 = jnp.dot(
            h * ns_ref[...], w2_ref[...], preferred_element_type=jnp.float32
        )
        t2_ref[0] = t2
        t2_ref[1] = t2

    return pl.pallas_call(
        body,
        out_shape=jax.ShapeDtypeStruct((NC, NROW, D_OUT), jnp.float32),
    )(p1, ns_col, nd_col, W1, b1r, W2)


def _tc_final(p2, nd_col, b2r):
    def body(p_ref, nd_ref, b2_ref, o_ref):
        o_ref[...] = (p_ref[0, :N] + p_ref[1, :N]) * nd_ref[:N] + b2_ref[...]

    return pl.pallas_call(
        body,
        out_shape=jax.ShapeDtypeStruct((N, D_OUT), jnp.float32),
    )(p2, nd_col, b2r)


def kernel(x, edge_index, W1, b1, W2, b2):
    src = edge_index[0]
    dst = edge_index[1]
    padi = jnp.full((EPAD - E,), N, jnp.int32)
    srcp = jnp.concatenate([src, padi])
    dstp = jnp.concatenate([dst, padi])
    x_pad = jnp.pad(x, ((0, NROW - N), (0, 0)))

    dst2d = dstp.reshape(EPAD // CH, CH)
    dst3d = dstp.reshape(EPAD // CH, 1, CH)

    degp = _sc_deg(srcp, dstp)
    dps = degp[:, :80, :].reshape(NC, NROW, 1)
    dpd = degp[:, HDST:HDST + 80, :].reshape(NC, NROW, 1)
    t1h, ns_col, nd_col = _tc_prep(x_pad, dps, dpd)

    p1 = _sc_agg(t1h, srcp, dst3d, False)
    t2s = _tc_mid(p1, ns_col, nd_col, W1, b1.reshape(1, D_HID), W2)

    p2 = _sc_agg(t2s, srcp, dst3d, True)
    return _tc_final(p2, nd_col, b2.reshape(1, D_OUT))


# speedup vs baseline: 10.6812x; 1.0003x over previous
import jax
import jax.numpy as jnp
from jax import lax
from jax.experimental import pallas as pl
from jax.experimental.pallas import tpu as pltpu
from jax.experimental.pallas import tpu_sc as plsc

N = 10000
E = 320000
D_IN = 128
D_HID = 128
D_OUT = 64

NROW = 10240
NC, NS = 2, 16
NW = NC * NS
CH = 128
NCHUNK = 80
EPW = CH * NCHUNK
EPAD = EPW * NW
HR = 256
HDST = 128
DEG_RPT = HR // NS


DEG0 = 13184
DEG1 = 2 * EPW - DEG0


def _deg_body(src_hbm, dst_hbm, out_hbm, sidx, didx, hist, ria, rib, zb, acc):
    cid = lax.axis_index("c")
    sid = lax.axis_index("s")
    zeros16 = jnp.zeros((16,), jnp.float32)
    ones16 = jnp.ones((16,), jnp.float32)
    iota16 = lax.iota(jnp.int32, 16)
    nedge = jnp.where(cid == 0, DEG0, DEG1)
    base = sid * 2 * EPW + cid * DEG0

    def zh(r, _):
        for c in range(8):
            hist[r, pl.ds(c * 16, 16)] = zeros16
        return 0

    lax.fori_loop(0, HR, zh, 0)

    for i in range(5):
        ria[pl.ds(i * 16, 16)] = iota16 + i * 16
        rib[pl.ds(i * 16, 16)] = iota16 + (HDST + i * 16)

    for r in range(DEG_RPT):
        for c in range(8):
            zb[r, pl.ds(c * 16, 16)] = zeros16
    pltpu.sync_copy(zb, acc.at[pl.ds(sid * DEG_RPT, DEG_RPT)])

    @pl.when(cid == 0)
    def _():
        pltpu.sync_copy(src_hbm.at[pl.ds(base, DEG0)], sidx.at[pl.ds(0, DEG0)])
        pltpu.sync_copy(dst_hbm.at[pl.ds(base, DEG0)], didx.at[pl.ds(0, DEG0)])

    @pl.when(cid == 1)
    def _():
        pltpu.sync_copy(src_hbm.at[pl.ds(base, DEG1)], sidx.at[pl.ds(0, DEG1)])
        pltpu.sync_copy(dst_hbm.at[pl.ds(base, DEG1)], didx.at[pl.ds(0, DEG1)])

    def step(i, _):
        s16 = sidx[pl.ds(i * 16, 16)]
        plsc.addupdate_scatter(
            hist,
            [lax.shift_right_logical(s16, 7), lax.bitwise_and(s16, 127)],
            ones16,
        )
        d16 = didx[pl.ds(i * 16, 16)]
        plsc.addupdate_scatter(
            hist,
            [lax.shift_right_logical(d16, 7) + HDST, lax.bitwise_and(d16, 127)],
            ones16,
        )
        return 0

    lax.fori_loop(0, nedge // 16, step, 0)

    plsc.subcore_barrier()
    pltpu.sync_copy(hist.at[pl.ds(0, 80)], acc.at[ria], add=True)
    pltpu.sync_copy(hist.at[pl.ds(HDST, 80)], acc.at[rib], add=True)
    plsc.subcore_barrier()

    sl = pl.ds(sid * DEG_RPT, DEG_RPT)
    pltpu.sync_copy(acc.at[sl], zb)
    pltpu.sync_copy(zb, out_hbm.at[cid, sl])


def _sc_deg(srcp, dstp):
    mesh = plsc.VectorSubcoreMesh(core_axis_name="c", subcore_axis_name="s")
    f = pl.kernel(
        _deg_body,
        out_type=jax.ShapeDtypeStruct((NC, HR, 128), jnp.float32),
        mesh=mesh,
        compiler_params=pltpu.CompilerParams(needs_layout_passes=False),
        scratch_types=[
            pltpu.VMEM((DEG0,), jnp.int32),
            pltpu.VMEM((DEG0,), jnp.int32),
            pltpu.VMEM((HR, 128), jnp.float32),
            pltpu.VMEM((80,), jnp.int32),
            pltpu.VMEM((80,), jnp.int32),
            pltpu.VMEM((DEG_RPT, 128), jnp.float32),
            pltpu.VMEM_SHARED((HR, 128), jnp.float32),
        ],
    )
    return f(srcp, dstp)


IXR = 8
DH = 64


def _make_sh_agg_body(nchk, half_edges):
    rpt = NROW // NS

    def body(tab_hbm, src_hbm, dst3d_hbm, out_hbm, sidxb, didxb, rows,
             tab_sh, acc, sem_i, sem_g, sem_s):
        cid = lax.axis_index("c")
        sid = lax.axis_index("s")
        zeros16 = jnp.zeros((16,), jnp.float32)
        wix = sid * NC + cid if half_edges else sid
        base = wix * nchk * CH

        def stage_idx(c, j):
            pltpu.async_copy(src_hbm.at[pl.ds(base + c * CH, CH)], sidxb[j],
                             sem_i[j])
            pltpu.async_copy(dst3d_hbm.at[wix * nchk + c], didxb[j],
                             sem_i[j])

        def wait_idx(j):
            pltpu.make_async_copy(src_hbm.at[pl.ds(base, CH)], sidxb[j],
                                  sem_i[j]).wait()
            pltpu.make_async_copy(dst3d_hbm.at[0], didxb[j],
                                  sem_i[j]).wait()

        def start_gather(j, b):
            pltpu.async_copy(tab_sh.at[sidxb[j]], rows[b], sem_g[b])

        def wait_gather(b):
            pltpu.make_async_copy(tab_sh.at[sidxb[0]], rows[b],
                                  sem_g[b]).wait()

        def start_scatter(j, b):
            pltpu.async_copy(rows[b], acc.at[didxb[j].at[0]], sem_s[b],
                             add=True)

        def wait_scatter(b):
            pltpu.make_async_copy(rows[b], acc.at[didxb[0].at[0]],
                                  sem_s[b]).wait()

        sl0 = pl.ds(sid * rpt, rpt)
        pltpu.sync_copy(tab_hbm.at[cid].at[sl0], tab_sh.at[sl0])
        for c in range(4):
            stage_idx(c, c)

        def zrow(r, _):
            for c in range(DH // 16):
                rows[1][r, pl.ds(c * 16, 16)] = zeros16
            return 0

        lax.fori_loop(0, CH, zrow, 0)
        for k in range(rpt // CH):
            pltpu.sync_copy(rows[1], acc.at[pl.ds(sid * rpt + k * CH, CH)])
        plsc.subcore_barrier()

        wait_idx(0)
        start_gather(0, 0)
        wait_idx(1)
        start_gather(1, 1)

        def outer(g, _):
            for b in range(IXR):
                c = g * IXR + b

                @pl.when(c + 4 < nchk)
                def _():
                    stage_idx(c + 4, (b + 4) % IXR)

                @pl.when(c >= 2)
                def _():
                    wait_scatter((b + 2) % 4)

                @pl.when(c + 2 < nchk)
                def _():
                    wait_idx((b + 2) % IXR)
                    start_gather((b + 2) % IXR, (b + 2) % 4)

                wait_gather(b % 4)
                start_scatter(b % IXR, b % 4)
            return 0

        lax.fori_loop(0, nchk // IXR, outer, 0)
        wait_scatter((nchk - 2) % 4)
        wait_scatter((nchk - 1) % 4)

        plsc.subcore_barrier()
        for k in range(rpt // CH):
            sl = pl.ds(sid * rpt + k * CH, CH)
            pltpu.sync_copy(acc.at[sl], rows[0])
            pltpu.sync_copy(rows[0], out_hbm.at[cid, sl])

    return body


def _sc_agg(tab2, srcp, dst3d, half_edges):
    nchk = NCHUNK if half_edges else 2 * NCHUNK
    mesh = plsc.VectorSubcoreMesh(core_axis_name="c", subcore_axis_name="s")
    f = pl.kernel(
        _make_sh_agg_body(nchk, half_edges),
        out_type=jax.ShapeDtypeStruct((NC, NROW, DH), jnp.float32),
        mesh=mesh,
        compiler_params=pltpu.CompilerParams(
            needs_layout_passes=False, use_tc_tiling_on_sc=False
        ),
        scratch_types=[
            [pltpu.VMEM((CH,), jnp.int32) for _ in range(IXR)],
            [pltpu.VMEM((1, CH), jnp.int32) for _ in range(IXR)],
            [pltpu.VMEM((CH, DH), jnp.float32) for _ in range(4)],
            pltpu.VMEM_SHARED((NROW, DH), jnp.float32),
            pltpu.VMEM_SHARED((NROW, DH), jnp.float32),
            [pltpu.SemaphoreType.DMA for _ in range(IXR)],
            [pltpu.SemaphoreType.DMA for _ in range(4)],
            [pltpu.SemaphoreType.DMA for _ in range(4)],
        ],
    )
    return f(tab2, srcp, dst3d)


def _tc_prep(x_pad, dps, dpd):
    def body(x_ref, dps_ref, dpd_ref, t1_ref, ns_ref, nd_ref):
        ds_ = dps_ref[0] + dps_ref[1]
        dd = dpd_ref[0] + dpd_ref[1]
        ns = jnp.where(ds_ > 0, lax.rsqrt(ds_), 0.0)
        nd = jnp.where(dd > 0, lax.rsqrt(dd), 0.0)
        ns_ref[...] = ns
        nd_ref[...] = nd
        t1 = x_ref[...] * ns
        t1_ref[0] = t1[:, :DH]
        t1_ref[1] = t1[:, DH:]

    return pl.pallas_call(
        body,
        out_shape=[
            jax.ShapeDtypeStruct((NC, NROW, DH), jnp.float32),
            jax.ShapeDtypeStruct((NROW, 1), jnp.float32),
            jax.ShapeDtypeStruct((NROW, 1), jnp.float32),
        ],
    )(x_pad, dps, dpd)


def _tc_mid(p1, ns_col, nd_col, W1, b1r, W2):
    def body(p_ref, ns_ref, nd_ref, w1_ref, b1_ref, w2_ref, t2_ref):
        nd = nd_ref[...]
        h = jnp.dot(p_ref[0] * nd, w1_ref[:DH],
                    preferred_element_type=jnp.float32)
        h = h + jnp.dot(p_ref[1] * nd, w1_ref[DH:],
                        preferred_element_type=jnp.float32)
        h = jnp.maximum(h + b1_ref[...], 0.0)
        t2 = jnp.dot(
            h * ns_ref[...], w2_ref[...], preferred_element_type=jnp.float32
        )
        t2_ref[0] = t2
        t2_ref[1] = t2

    return pl.pallas_call(
        body,
        out_shape=jax.ShapeDtypeStruct((NC, NROW, D_OUT), jnp.float32),
    )(p1, ns_col, nd_col, W1, b1r, W2)


def _tc_final(p2, nd_col, b2r):
    def body(p_ref, nd_ref, b2_ref, o_ref):
        o_ref[...] = (p_ref[0, :N] + p_ref[1, :N]) * nd_ref[:N] + b2_ref[...]

    return pl.pallas_call(
        body,
        out_shape=jax.ShapeDtypeStruct((N, D_OUT), jnp.float32),
    )(p2, nd_col, b2r)


def kernel(x, edge_index, W1, b1, W2, b2):
    src = edge_index[0]
    dst = edge_index[1]
    padi = jnp.full((EPAD - E,), N, jnp.int32)
    srcp = jnp.concatenate([src, padi])
    dstp = jnp.concatenate([dst, padi])
    x_pad = jnp.pad(x, ((0, NROW - N), (0, 0)))

    dst3d = dstp.reshape(EPAD // CH, 1, CH)

    degp = _sc_deg(srcp, dstp)
    dps = degp[:, :80, :].reshape(NC, NROW, 1)
    dpd = degp[:, HDST:HDST + 80, :].reshape(NC, NROW, 1)
    t1h, ns_col, nd_col = _tc_prep(x_pad, dps, dpd)

    p1 = _sc_agg(t1h, srcp, dst3d, False)
    t2s = _tc_mid(p1, ns_col, nd_col, W1, b1.reshape(1, D_HID), W2)

    p2 = _sc_agg(t2s, srcp, dst3d, True)
    return _tc_final(p2, nd_col, b2.reshape(1, D_OUT))
